# baseline (speedup 1.0000x reference)
import functools

import jax
import jax.numpy as jnp
from jax import lax
from jax.experimental import pallas as pl
from jax.experimental.pallas import tpu as pltpu
from jax.experimental.pallas import tpu_sc as plsc

_N = 131072
_DIM = 2
_E = 8
_H = 64

_NW = 32
_CHUNK = _N // _NW
_GRP = 128
_NGRP = _CHUNK // _GRP

_TILE = 2048
_TMAX = _N // _TILE + 16
_P = _TMAX * _TILE


_SUB = 128


def _router_body(x_ref, wr_ref, br_ref, logits_ref, best_ref, rank_ref,
                 hist_ref):
  x = x_ref[...]
  logits = (jnp.dot(x, wr_ref[...].T, preferred_element_type=jnp.float32)
            + br_ref[...])
  logits_ref[...] = logits
  best = jnp.argmax(logits, axis=-1).astype(jnp.int32)
  onehot = (best[:, None]
            == lax.broadcasted_iota(jnp.int32, (1, _E), 1)).astype(jnp.float32)
  counts = jnp.sum(onehot, axis=0)
  hist_ref[...] = jnp.concatenate(
      [counts.astype(jnp.int32), jnp.zeros((16 - _E,), jnp.int32)]
  ).reshape(1, 1, 16)
  best_ref[...] = best.reshape(1, 1, _CHUNK)
  tri = (lax.broadcasted_iota(jnp.int32, (_SUB, _SUB), 0)
         >= lax.broadcasted_iota(jnp.int32, (_SUB, _SUB), 1)
         ).astype(jnp.float32)
  running = jnp.zeros((1, _E), jnp.float32)
  parts = []
  for i in range(_CHUNK // _SUB):
    blk = onehot[i * _SUB:(i + 1) * _SUB]
    s = jnp.dot(tri, blk, preferred_element_type=jnp.float32)
    r = jnp.sum((s + running - 1.0) * blk, axis=1)
    parts.append(r.reshape(1, 1, _SUB))
    running = running + jnp.sum(blk, axis=0, keepdims=True)
  rank_ref[...] = jnp.concatenate(parts, axis=2).astype(jnp.int32)


def _router(x, Wr, br):
  full = lambda *s: pl.BlockSpec(s, lambda i: (0,) * len(s))
  return pl.pallas_call(
      _router_body,
      grid=(_NW,),
      in_specs=[
          pl.BlockSpec((_CHUNK, _DIM), lambda i: (i, 0)),
          full(_E, _DIM), full(_E),
      ],
      out_specs=[
          pl.BlockSpec((_CHUNK, _E), lambda i: (i, 0)),
          pl.BlockSpec((1, 1, _CHUNK), lambda i: (i, 0, 0)),
          pl.BlockSpec((1, 1, _CHUNK), lambda i: (i, 0, 0)),
          pl.BlockSpec((1, 1, 16), lambda i: (i, 0, 0)),
      ],
      out_shape=[
          jax.ShapeDtypeStruct((_N, _E), jnp.float32),
          jax.ShapeDtypeStruct((_NW, 1, _CHUNK), jnp.int32),
          jax.ShapeDtypeStruct((_NW, 1, _CHUNK), jnp.int32),
          jax.ShapeDtypeStruct((_NW, 1, 16), jnp.int32),
      ],
      compiler_params=pltpu.CompilerParams(
          dimension_semantics=("parallel",)),
  )(x, Wr, br)


def _dispatch_body(x0_hbm, x1_hbm, best_hbm, rank_hbm, hist_hbm,
                   xs0_hbm, xs1_hbm, dest_hbm, tmap_hbm,
                   x0_v, x1_v, best_v, rank_v, hist_v, base_v, dest_v,
                   map_v, sem):
  nc = 2
  wid = lax.axis_index("s") * nc + lax.axis_index("c")
  base = wid * _CHUNK
  lanes = lax.iota(jnp.int32, 16)

  pltpu.sync_copy(x0_hbm.at[pl.ds(base, _CHUNK)], x0_v)
  pltpu.sync_copy(x1_hbm.at[pl.ds(base, _CHUNK)], x1_v)
  pltpu.sync_copy(best_hbm.at[wid, 0], best_v)
  pltpu.sync_copy(rank_hbm.at[wid, 0], rank_v)
  pltpu.sync_copy(hist_hbm, hist_v)

  wmask_a = lanes < wid
  wmask_b = (lanes + 16) < wid
  base_acc = jnp.zeros((16,), jnp.int32)
  tot_acc = jnp.zeros((16,), jnp.int32)
  for e in range(_E):
    fe = jnp.full((16,), e, jnp.int32)
    va = plsc.load_gather(hist_v, [lanes * 16 + fe])
    vb = plsc.load_gather(hist_v, [(lanes + 16) * 16 + fe])
    pre = (jnp.sum(jnp.where(wmask_a, va, 0))
           + jnp.sum(jnp.where(wmask_b, vb, 0)))
    tot = jnp.sum(va) + jnp.sum(vb)
    sel = lanes == e
    base_acc = jnp.where(sel, pre, base_acc)
    tot_acc = jnp.where(sel, tot, tot_acc)

  tiles = (tot_acc + (_TILE - 1)) // _TILE
  starts_t = plsc.cumsum(tiles) - tiles
  base_v[...] = starts_t * _TILE + base_acc

  @pl.when(wid == 0)
  def _():
    for j in range(_TMAX // 16):
      tv = lanes + 16 * j
      acc = jnp.zeros((16,), jnp.int32)
      for e in range(_E):
        st_e = jnp.sum(jnp.where(lanes == e, starts_t, 0))
        acc = acc + (tv >= st_e).astype(jnp.int32)
      map_v[pl.ds(16 * j, 16)] = acc - 1
    pltpu.sync_copy(map_v, tmap_hbm)

  copies = []
  for g in range(_NGRP):
    def _slots(k, _):
      ev = best_v[pl.ds(g * _GRP + k * 16, 16)]
      rv = rank_v[pl.ds(g * _GRP + k * 16, 16)]
      dest_v[g, pl.ds(k * 16, 16)] = plsc.load_gather(base_v, [ev]) + rv
      return _
    lax.fori_loop(0, _GRP // 16, _slots, None)
    copies.append(pltpu.async_copy(
        x0_v.at[pl.ds(g * _GRP, _GRP)], xs0_hbm.at[dest_v.at[g]], sem))
    copies.append(pltpu.async_copy(
        x1_v.at[pl.ds(g * _GRP, _GRP)], xs1_hbm.at[dest_v.at[g]], sem))
  for c in copies:
    c.wait()
  pltpu.sync_copy(dest_v, dest_hbm.at[wid])


def _dispatch(x0, x1, best, rank, hist):
  mesh = plsc.VectorSubcoreMesh(core_axis_name="c", subcore_axis_name="s")
  return pl.kernel(
      _dispatch_body,
      out_type=[
          jax.ShapeDtypeStruct((_P,), jnp.float32),
          jax.ShapeDtypeStruct((_P,), jnp.float32),
          jax.ShapeDtypeStruct((_NW, _NGRP, _GRP), jnp.int32),
          jax.ShapeDtypeStruct((_TMAX,), jnp.int32),
      ],
      mesh=mesh,
      scratch_types=[
          pltpu.VMEM((_CHUNK,), jnp.float32),
          pltpu.VMEM((_CHUNK,), jnp.float32),
          pltpu.VMEM((_CHUNK,), jnp.int32),
          pltpu.VMEM((_CHUNK,), jnp.int32),
          pltpu.VMEM((_NW * 16,), jnp.int32),
          pltpu.VMEM((16,), jnp.int32),
          pltpu.VMEM((_NGRP, _GRP), jnp.int32),
          pltpu.VMEM((_TMAX,), jnp.int32),
          pltpu.SemaphoreType.DMA,
      ],
      compiler_params=pltpu.CompilerParams(needs_layout_passes=False),
  )(x0, x1, best, rank, hist)


def _mlp_body(tmap_ref, xs0_ref, xs1_ref, w1_ref, b1_ref, w2_ref, b2_ref,
              w3_ref, b3_ref, ys0_ref, ys1_ref):
  del tmap_ref
  xt = jnp.concatenate([xs0_ref[0], xs1_ref[0]], axis=0)
  h1 = jnp.maximum(
      jnp.dot(w1_ref[0], xt, preferred_element_type=jnp.float32)
      + b1_ref[0], 0.0)
  h2 = jnp.maximum(
      jnp.dot(w2_ref[0], h1, preferred_element_type=jnp.float32)
      + b2_ref[0], 0.0)
  yt = (jnp.dot(w3_ref[0], h2, preferred_element_type=jnp.float32)
        + b3_ref[0])
  ys0_ref[0] = yt[0:1, :]
  ys1_ref[0] = yt[1:2, :]


def _mlp(tmap, xs0, xs1, W1, b1, W2, b2, W3, b3):
  em3 = lambda t, m: (m[t], 0, 0)
  tok = lambda t, m: (t, 0, 0)
  grid_spec = pltpu.PrefetchScalarGridSpec(
      num_scalar_prefetch=1,
      grid=(_TMAX,),
      in_specs=[
          pl.BlockSpec((1, 1, _TILE), tok),
          pl.BlockSpec((1, 1, _TILE), tok),
          pl.BlockSpec((1, _H, _DIM), em3),
          pl.BlockSpec((1, _H, 1), em3),
          pl.BlockSpec((1, _H, _H), em3),
          pl.BlockSpec((1, _H, 1), em3),
          pl.BlockSpec((1, _DIM, _H), em3),
          pl.BlockSpec((1, _DIM, 1), em3),
      ],
      out_specs=[
          pl.BlockSpec((1, 1, _TILE), tok),
          pl.BlockSpec((1, 1, _TILE), tok),
      ],
  )
  return pl.pallas_call(
      _mlp_body,
      grid_spec=grid_spec,
      out_shape=[
          jax.ShapeDtypeStruct((_TMAX, 1, _TILE), jnp.float32),
          jax.ShapeDtypeStruct((_TMAX, 1, _TILE), jnp.float32),
      ],
      compiler_params=pltpu.CompilerParams(
          dimension_semantics=("arbitrary",)),
  )(tmap, xs0.reshape(_TMAX, 1, _TILE), xs1.reshape(_TMAX, 1, _TILE),
    W1, b1.reshape(_E, _H, 1), W2, b2.reshape(_E, _H, 1),
    W3, b3.reshape(_E, _DIM, 1))


def _collect_body(ys0_hbm, ys1_hbm, dest_hbm, out0_hbm, out1_hbm,
                  dest_v, y0_v, y1_v, sem):
  nc = 2
  wid = lax.axis_index("s") * nc + lax.axis_index("c")
  pltpu.sync_copy(dest_hbm.at[wid], dest_v)
  copies = []
  for g in range(_NGRP):
    copies.append(pltpu.async_copy(
        ys0_hbm.at[dest_v.at[g]], y0_v.at[pl.ds(g * _GRP, _GRP)], sem))
    copies.append(pltpu.async_copy(
        ys1_hbm.at[dest_v.at[g]], y1_v.at[pl.ds(g * _GRP, _GRP)], sem))
  for c in copies:
    c.wait()
  pltpu.sync_copy(y0_v, out0_hbm.at[pl.ds(wid * _CHUNK, _CHUNK)])
  pltpu.sync_copy(y1_v, out1_hbm.at[pl.ds(wid * _CHUNK, _CHUNK)])


def _collect(ys0, ys1, dest):
  mesh = plsc.VectorSubcoreMesh(core_axis_name="c", subcore_axis_name="s")
  return pl.kernel(
      _collect_body,
      out_type=[
          jax.ShapeDtypeStruct((_N,), jnp.float32),
          jax.ShapeDtypeStruct((_N,), jnp.float32),
      ],
      mesh=mesh,
      scratch_types=[
          pltpu.VMEM((_NGRP, _GRP), jnp.int32),
          pltpu.VMEM((_CHUNK,), jnp.float32),
          pltpu.VMEM((_CHUNK,), jnp.float32),
          pltpu.SemaphoreType.DMA,
      ],
      compiler_params=pltpu.CompilerParams(needs_layout_passes=False),
  )(ys0, ys1, dest)


@jax.jit
def kernel(x, Wr, br, W1, b1, W2, b2, W3, b3):
  logits, best, rank, hist = _router(x, Wr, br)
  xs0, xs1, dest, tmap = _dispatch(x[:, 0], x[:, 1], best, rank,
                                   hist.reshape(_NW * 16))
  ys0, ys1 = _mlp(tmap, xs0, xs1, W1, b1, W2, b2, W3, b3)
  out0, out1 = _collect(ys0.reshape(_P), ys1.reshape(_P), dest)
  return jnp.stack([out0, out1], axis=1), logits


# speedup vs baseline: 2.6242x; 2.6242x over previous
import functools

import jax
import jax.numpy as jnp
from jax import lax
from jax.experimental import pallas as pl
from jax.experimental.pallas import tpu as pltpu
from jax.experimental.pallas import tpu_sc as plsc

_N = 131072
_DIM = 2
_E = 8
_H = 64

_NW = 32
_CHUNK = _N // _NW
_GRP = 128
_NGRP = _CHUNK // _GRP

_TILE = 2048
_TMAX = _N // _TILE + 16
_P = _TMAX * _TILE


_SUB = 128


def _router_body(x_ref, wr_ref, br_ref, logits_ref, best_ref, rank_ref,
                 hist_ref):
  x = x_ref[...]
  logits = (jnp.dot(x, wr_ref[...].T, preferred_element_type=jnp.float32)
            + br_ref[...])
  logits_ref[...] = logits
  best = jnp.argmax(logits, axis=-1).astype(jnp.int32)
  onehot = (best[:, None]
            == lax.broadcasted_iota(jnp.int32, (1, _E), 1)).astype(jnp.float32)
  counts = jnp.sum(onehot, axis=0)
  hist_ref[...] = jnp.concatenate(
      [counts.astype(jnp.int32), jnp.zeros((16 - _E,), jnp.int32)]
  ).reshape(1, 1, 16)
  best_ref[...] = best.reshape(1, 1, _CHUNK)
  tri = (lax.broadcasted_iota(jnp.int32, (_SUB, _SUB), 0)
         >= lax.broadcasted_iota(jnp.int32, (_SUB, _SUB), 1)
         ).astype(jnp.float32)
  running = jnp.zeros((1, _E), jnp.float32)
  parts = []
  for i in range(_CHUNK // _SUB):
    blk = onehot[i * _SUB:(i + 1) * _SUB]
    s = jnp.dot(tri, blk, preferred_element_type=jnp.float32)
    r = jnp.sum((s + running - 1.0) * blk, axis=1)
    parts.append(r.reshape(1, 1, _SUB))
    running = running + jnp.sum(blk, axis=0, keepdims=True)
  rank_ref[...] = jnp.concatenate(parts, axis=2).astype(jnp.int32)


def _router(x, Wr, br):
  full = lambda *s: pl.BlockSpec(s, lambda i: (0,) * len(s))
  return pl.pallas_call(
      _router_body,
      grid=(_NW,),
      in_specs=[
          pl.BlockSpec((_CHUNK, _DIM), lambda i: (i, 0)),
          full(_E, _DIM), full(_E),
      ],
      out_specs=[
          pl.BlockSpec((_CHUNK, _E), lambda i: (i, 0)),
          pl.BlockSpec((1, 1, _CHUNK), lambda i: (i, 0, 0)),
          pl.BlockSpec((1, 1, _CHUNK), lambda i: (i, 0, 0)),
          pl.BlockSpec((1, 1, 16), lambda i: (i, 0, 0)),
      ],
      out_shape=[
          jax.ShapeDtypeStruct((_N, _E), jnp.float32),
          jax.ShapeDtypeStruct((_NW, 1, _CHUNK), jnp.int32),
          jax.ShapeDtypeStruct((_NW, 1, _CHUNK), jnp.int32),
          jax.ShapeDtypeStruct((_NW, 1, 16), jnp.int32),
      ],
      compiler_params=pltpu.CompilerParams(
          dimension_semantics=("parallel",)),
  )(x, Wr, br)


def _dispatch_body(x0_hbm, x1_hbm, best_hbm, rank_hbm, hist_hbm,
                   xs0_hbm, xs1_hbm, dest_hbm, tmap_hbm,
                   x0_v, x1_v, best_v, rank_v, hist_v, base_v, dest_v,
                   map_v, zero_v, xsh0, xsh1, sem):
  nc = 2
  cid = lax.axis_index("c")
  sid = lax.axis_index("s")
  wid = sid * nc + cid
  base = wid * _CHUNK
  lanes = lax.iota(jnp.int32, 16)

  zslice = _P // 16
  zlen = zslice // 2

  def _z(i, _):
    zero_v[pl.ds(i * 16, 16)] = jnp.zeros((16,), jnp.float32)
    return _
  lax.fori_loop(0, zlen // 16, _z, None)
  for m in range(2):
    pltpu.sync_copy(zero_v, xsh0.at[pl.ds(sid * zslice + m * zlen, zlen)])
    pltpu.sync_copy(zero_v, xsh1.at[pl.ds(sid * zslice + m * zlen, zlen)])

  pltpu.sync_copy(x0_hbm.at[pl.ds(base, _CHUNK)], x0_v)
  pltpu.sync_copy(x1_hbm.at[pl.ds(base, _CHUNK)], x1_v)
  pltpu.sync_copy(best_hbm.at[wid, 0], best_v)
  pltpu.sync_copy(rank_hbm.at[wid, 0], rank_v)
  pltpu.sync_copy(hist_hbm, hist_v)

  wmask_a = lanes < wid
  wmask_b = (lanes + 16) < wid
  base_acc = jnp.zeros((16,), jnp.int32)
  tot_acc = jnp.zeros((16,), jnp.int32)
  for e in range(_E):
    fe = jnp.full((16,), e, jnp.int32)
    va = plsc.load_gather(hist_v, [lanes * 16 + fe])
    vb = plsc.load_gather(hist_v, [(lanes + 16) * 16 + fe])
    pre = (jnp.sum(jnp.where(wmask_a, va, 0))
           + jnp.sum(jnp.where(wmask_b, vb, 0)))
    tot = jnp.sum(va) + jnp.sum(vb)
    sel = lanes == e
    base_acc = jnp.where(sel, pre, base_acc)
    tot_acc = jnp.where(sel, tot, tot_acc)

  tiles = (tot_acc + (_TILE - 1)) // _TILE
  starts_t = plsc.cumsum(tiles) - tiles
  base_v[...] = starts_t * _TILE + base_acc

  @pl.when(wid == 0)
  def _():
    for j in range(_TMAX // 16):
      tv = lanes + 16 * j
      acc = jnp.zeros((16,), jnp.int32)
      for e in range(_E):
        st_e = jnp.sum(jnp.where(lanes == e, starts_t, 0))
        acc = acc + (tv >= st_e).astype(jnp.int32)
      map_v[pl.ds(16 * j, 16)] = acc - 1
    pltpu.sync_copy(map_v, tmap_hbm)

  plsc.subcore_barrier()
  copies = []
  for g in range(_NGRP):
    def _slots(k, _):
      ev = best_v[pl.ds(g * _GRP + k * 16, 16)]
      rv = rank_v[pl.ds(g * _GRP + k * 16, 16)]
      dest_v[g, pl.ds(k * 16, 16)] = plsc.load_gather(base_v, [ev]) + rv
      return _
    lax.fori_loop(0, _GRP // 16, _slots, None)
    copies.append(pltpu.async_copy(
        x0_v.at[pl.ds(g * _GRP, _GRP)], xsh0.at[dest_v.at[g]], sem,
        add=True))
    copies.append(pltpu.async_copy(
        x1_v.at[pl.ds(g * _GRP, _GRP)], xsh1.at[dest_v.at[g]], sem,
        add=True))
  for c in copies:
    c.wait()
  pltpu.sync_copy(dest_v, dest_hbm.at[wid])
  plsc.subcore_barrier()

  @pl.when(sid == 0)
  def _():
    pltpu.sync_copy(xsh0, xs0_hbm.at[cid])
    pltpu.sync_copy(xsh1, xs1_hbm.at[cid])


def _dispatch(x0, x1, best, rank, hist):
  mesh = plsc.VectorSubcoreMesh(core_axis_name="c", subcore_axis_name="s")
  return pl.kernel(
      _dispatch_body,
      out_type=[
          jax.ShapeDtypeStruct((2, _P), jnp.float32),
          jax.ShapeDtypeStruct((2, _P), jnp.float32),
          jax.ShapeDtypeStruct((_NW, _NGRP, _GRP), jnp.int32),
          jax.ShapeDtypeStruct((_TMAX,), jnp.int32),
      ],
      mesh=mesh,
      scratch_types=[
          pltpu.VMEM((_CHUNK,), jnp.float32),
          pltpu.VMEM((_CHUNK,), jnp.float32),
          pltpu.VMEM((_CHUNK,), jnp.int32),
          pltpu.VMEM((_CHUNK,), jnp.int32),
          pltpu.VMEM((_NW * 16,), jnp.int32),
          pltpu.VMEM((16,), jnp.int32),
          pltpu.VMEM((_NGRP, _GRP), jnp.int32),
          pltpu.VMEM((_TMAX,), jnp.int32),
          pltpu.VMEM((_P // 32,), jnp.float32),
          pltpu.VMEM_SHARED((_P,), jnp.float32),
          pltpu.VMEM_SHARED((_P,), jnp.float32),
          pltpu.SemaphoreType.DMA,
      ],
      compiler_params=pltpu.CompilerParams(needs_layout_passes=False),
  )(x0, x1, best, rank, hist)


def _mlp_body(tmap_ref, xs0a_ref, xs0b_ref, xs1a_ref, xs1b_ref,
              w1_ref, b1_ref, w2_ref, b2_ref,
              w3_ref, b3_ref, ys0_ref, ys1_ref):
  del tmap_ref
  xt = jnp.concatenate([xs0a_ref[0] + xs0b_ref[0],
                        xs1a_ref[0] + xs1b_ref[0]], axis=0)
  h1 = jnp.maximum(
      jnp.dot(w1_ref[0], xt, preferred_element_type=jnp.float32)
      + b1_ref[0], 0.0)
  h2 = jnp.maximum(
      jnp.dot(w2_ref[0], h1, preferred_element_type=jnp.float32)
      + b2_ref[0], 0.0)
  yt = (jnp.dot(w3_ref[0], h2, preferred_element_type=jnp.float32)
        + b3_ref[0])
  ys0_ref[0] = yt[0:1, :]
  ys1_ref[0] = yt[1:2, :]


def _mlp(tmap, xs0, xs1, W1, b1, W2, b2, W3, b3):
  em3 = lambda t, m: (m[t], 0, 0)
  tok = lambda t, m: (t, 0, 0)
  grid_spec = pltpu.PrefetchScalarGridSpec(
      num_scalar_prefetch=1,
      grid=(_TMAX,),
      in_specs=[
          pl.BlockSpec((1, 1, _TILE), tok),
          pl.BlockSpec((1, 1, _TILE), tok),
          pl.BlockSpec((1, 1, _TILE), tok),
          pl.BlockSpec((1, 1, _TILE), tok),
          pl.BlockSpec((1, _H, _DIM), em3),
          pl.BlockSpec((1, _H, 1), em3),
          pl.BlockSpec((1, _H, _H), em3),
          pl.BlockSpec((1, _H, 1), em3),
          pl.BlockSpec((1, _DIM, _H), em3),
          pl.BlockSpec((1, _DIM, 1), em3),
      ],
      out_specs=[
          pl.BlockSpec((1, 1, _TILE), tok),
          pl.BlockSpec((1, 1, _TILE), tok),
      ],
  )
  return pl.pallas_call(
      _mlp_body,
      grid_spec=grid_spec,
      out_shape=[
          jax.ShapeDtypeStruct((_TMAX, 1, _TILE), jnp.float32),
          jax.ShapeDtypeStruct((_TMAX, 1, _TILE), jnp.float32),
      ],
      compiler_params=pltpu.CompilerParams(
          dimension_semantics=("arbitrary",)),
  )(tmap,
    xs0[0].reshape(_TMAX, 1, _TILE), xs0[1].reshape(_TMAX, 1, _TILE),
    xs1[0].reshape(_TMAX, 1, _TILE), xs1[1].reshape(_TMAX, 1, _TILE),
    W1, b1.reshape(_E, _H, 1), W2, b2.reshape(_E, _H, 1),
    W3, b3.reshape(_E, _DIM, 1))


def _collect_body(ys0_hbm, ys1_hbm, dest_hbm, out0_hbm, out1_hbm,
                  dest_v, y0_v, y1_v, sem):
  nc = 2
  wid = lax.axis_index("s") * nc + lax.axis_index("c")
  pltpu.sync_copy(dest_hbm.at[wid], dest_v)
  copies = []
  for g in range(_NGRP):
    copies.append(pltpu.async_copy(
        ys0_hbm.at[dest_v.at[g]], y0_v.at[pl.ds(g * _GRP, _GRP)], sem))
    copies.append(pltpu.async_copy(
        ys1_hbm.at[dest_v.at[g]], y1_v.at[pl.ds(g * _GRP, _GRP)], sem))
  for c in copies:
    c.wait()
  pltpu.sync_copy(y0_v, out0_hbm.at[pl.ds(wid * _CHUNK, _CHUNK)])
  pltpu.sync_copy(y1_v, out1_hbm.at[pl.ds(wid * _CHUNK, _CHUNK)])


def _collect(ys0, ys1, dest):
  mesh = plsc.VectorSubcoreMesh(core_axis_name="c", subcore_axis_name="s")
  return pl.kernel(
      _collect_body,
      out_type=[
          jax.ShapeDtypeStruct((_N,), jnp.float32),
          jax.ShapeDtypeStruct((_N,), jnp.float32),
      ],
      mesh=mesh,
      scratch_types=[
          pltpu.VMEM((_NGRP, _GRP), jnp.int32),
          pltpu.VMEM((_CHUNK,), jnp.float32),
          pltpu.VMEM((_CHUNK,), jnp.float32),
          pltpu.SemaphoreType.DMA,
      ],
      compiler_params=pltpu.CompilerParams(needs_layout_passes=False),
  )(ys0, ys1, dest)


@jax.jit
def kernel(x, Wr, br, W1, b1, W2, b2, W3, b3):
  logits, best, rank, hist = _router(x, Wr, br)
  xs0, xs1, dest, tmap = _dispatch(x[:, 0], x[:, 1], best, rank,
                                   hist.reshape(_NW * 16))
  ys0, ys1 = _mlp(tmap, xs0, xs1, W1, b1, W2, b2, W3, b3)
  out0, out1 = _collect(ys0.reshape(_P), ys1.reshape(_P), dest)
  return jnp.stack([out0, out1], axis=1), logits
